# Initial kernel scaffold; baseline (speedup 1.0000x reference)
import jax
import jax.numpy as jnp
from jax.experimental import pallas as pl


def kernel(walk, negative, emb):
    raise NotImplementedError("write your pallas kernel here")


# speedup vs baseline: 32.2074x; 32.2074x over previous
import functools

import jax
import jax.numpy as jnp
from jax import lax
from jax.experimental import pallas as pl
from jax.experimental.pallas import tpu as pltpu
from jax.experimental.pallas import tpu_sc as plsc

DIM = 64
L = 50
K = 5
M = 10
B = 4096
LK = L - K
POS_N = LK * K
NEG_N = LK * M
NEG_C = 5
NEG_W = NEG_N // NEG_C
OUT_W = LK * 16

NUM_CORES = 2
NUM_SUBCORES = 16
NW = NUM_CORES * NUM_SUBCORES
WPB = B // NW
IDX_CHUNK = 16


def _dot_chunks(w_chunks, row_ref, r):
    acc = w_chunks[0] * row_ref[r, pl.ds(0, 16)]
    for c in range(1, 4):
        acc = acc + w_chunks[c] * row_ref[r, pl.ds(c * 16, 16)]
    return jnp.sum(acc)


def _sc_body(walk_hbm, neg_hbm, emb_hbm, out_hbm,
             widx_v, nidx_v, wrows_v, nrows_v, obuf_v, sem):
    cid = lax.axis_index("c")
    sid = lax.axis_index("s")
    wid = sid * NUM_CORES + cid
    base = wid * WPB

    def walk_chunk(ci, carry):
        b0 = base + ci * IDX_CHUNK
        pltpu.sync_copy(walk_hbm.at[pl.ds(b0, IDX_CHUNK)], widx_v)
        pltpu.sync_copy(neg_hbm.at[pl.ds(b0, IDX_CHUNK)], nidx_v)

        def one_walk(j, carry2):
            cps = [pltpu.async_copy(emb_hbm.at[widx_v.at[j]], wrows_v, sem)]
            for c in range(NEG_C):
                cps.append(pltpu.async_copy(
                    emb_hbm.at[nidx_v.at[j, c]],
                    nrows_v.at[pl.ds(c * NEG_W, NEG_W)], sem))
            for cp in cps:
                cp.wait()

            lanes = lax.iota(jnp.int32, 16)

            def one_pos(i, carry3):
                w = [wrows_v[i, pl.ds(c * 16, 16)] for c in range(4)]
                res = jnp.zeros((16,), jnp.float32)
                for k in range(K):
                    s = _dot_chunks(w, wrows_v, i + 1 + k)
                    res = jnp.where(lanes == k, jnp.full((16,), s, jnp.float32), res)
                for m in range(M):
                    s = _dot_chunks(w, nrows_v, i * M + m)
                    res = jnp.where(lanes == K + m, jnp.full((16,), s, jnp.float32), res)
                obuf_v[i] = res
                return carry3

            lax.fori_loop(0, LK, one_pos, 0)
            pltpu.sync_copy(obuf_v, out_hbm.at[b0 + j])
            return carry2

        lax.fori_loop(0, IDX_CHUNK, one_walk, 0)
        return carry

    lax.fori_loop(0, WPB // IDX_CHUNK, walk_chunk, 0)


@jax.jit
def _sc_call(walk, neg_r, emb):
    mesh = plsc.VectorSubcoreMesh(
        core_axis_name="c", subcore_axis_name="s",
        num_cores=NUM_CORES, num_subcores=NUM_SUBCORES)
    return pl.kernel(
        _sc_body,
        out_type=jax.ShapeDtypeStruct((B, LK, 16), jnp.float32),
        mesh=mesh,
        compiler_params=pltpu.CompilerParams(
            needs_layout_passes=False, use_tc_tiling_on_sc=False),
        scratch_types=[
            pltpu.VMEM((IDX_CHUNK, L), jnp.int32),
            pltpu.VMEM((IDX_CHUNK, NEG_C, NEG_W), jnp.int32),
            pltpu.VMEM((L, DIM), jnp.float32),
            pltpu.VMEM((NEG_N, DIM), jnp.float32),
            pltpu.VMEM((LK, 16), jnp.float32),
            pltpu.SemaphoreType.DMA,
        ],
    )(walk, neg_r, emb)


def kernel(walk, negative, emb):
    neg_r = negative.reshape(B, NEG_C, NEG_W)
    out = _sc_call(walk, neg_r, emb)
    pos = out[:, :, :K]
    neg = out[:, :, K:K + M]
    return pos, neg


# speedup vs baseline: 38.7254x; 1.2024x over previous
import jax
import jax.numpy as jnp
from jax import lax
from jax.experimental import pallas as pl
from jax.experimental.pallas import tpu as pltpu
from jax.experimental.pallas import tpu_sc as plsc

DIM = 64
L = 50
K = 5
M = 10
B = 4096
LK = L - K
NEG_N = LK * M
ROWS = L + NEG_N
GC = 5
GW = ROWS // GC

NUM_CORES = 2
NUM_SUBCORES = 16
NW = NUM_CORES * NUM_SUBCORES
WPB = B // NW
IDX_CHUNK = 16


def _dot_chunks(w_chunks, row_ref, r):
    acc = w_chunks[0] * row_ref[r, pl.ds(0, 16)]
    for c in range(1, 4):
        acc = acc + w_chunks[c] * row_ref[r, pl.ds(c * 16, 16)]
    return jnp.sum(acc)


def _sc_body(idx_hbm, emb_hbm, out_hbm,
             idx_v, rows0_v, rows1_v, obuf0_v, obuf1_v,
             gsem0, gsem1, osem0, osem1):
    cid = lax.axis_index("c")
    sid = lax.axis_index("s")
    wid = sid * NUM_CORES + cid
    base = wid * WPB

    def stage(chunk):
        pltpu.sync_copy(
            idx_hbm.at[pl.ds(base + chunk * IDX_CHUNK, IDX_CHUNK)],
            idx_v.at[chunk % 2])

    def gather_descs(x, rowbuf, gsem):
        cpx = (x // IDX_CHUNK) % 2
        wj = x % IDX_CHUNK
        return [
            pltpu.make_async_copy(
                emb_hbm.at[idx_v.at[cpx, wj, c]],
                rowbuf.at[pl.ds(c * GW, GW)], gsem)
            for c in range(GC)
        ]

    def issue(x, rowbuf, gsem):
        for d in gather_descs(x, rowbuf, gsem):
            d.start()

    def drain(x, rowbuf, gsem):
        for d in gather_descs(x, rowbuf, gsem):
            d.wait()

    lanes = lax.iota(jnp.int32, 16)

    def compute(rowbuf, obuf):
        def one_pos(i, carry):
            w = [rowbuf[i, pl.ds(c * 16, 16)] for c in range(4)]
            res = jnp.zeros((16,), jnp.float32)
            for k in range(K):
                s = _dot_chunks(w, rowbuf, i + 1 + k)
                res = jnp.where(lanes == k, jnp.full((16,), s, jnp.float32), res)
            for m in range(M):
                s = _dot_chunks(w, rowbuf, L + i * M + m)
                res = jnp.where(lanes == K + m, jnp.full((16,), s, jnp.float32), res)
            obuf[i] = res
            return carry

        lax.fori_loop(0, LK, one_pos, 0)

    def slot(x, rowbuf, obuf, gsem, osem):
        @pl.when(x >= 2)
        def _():
            pltpu.make_async_copy(obuf, out_hbm.at[base + x - 2], osem).wait()

        drain(x, rowbuf, gsem)
        compute(rowbuf, obuf)
        pltpu.async_copy(obuf, out_hbm.at[base + x], osem)

        nxt = x + 2

        @pl.when(nxt < WPB)
        def _():
            @pl.when(nxt % IDX_CHUNK == 0)
            def _():
                stage(nxt // IDX_CHUNK)

            issue(nxt, rowbuf, gsem)

    stage(0)
    issue(0, rows0_v, gsem0)
    issue(1, rows1_v, gsem1)

    @pl.loop(0, WPB, step=2)
    def _iter(a):
        slot(a, rows0_v, obuf0_v, gsem0, osem0)
        slot(a + 1, rows1_v, obuf1_v, gsem1, osem1)

    pltpu.make_async_copy(obuf0_v, out_hbm.at[base + WPB - 2], osem0).wait()
    pltpu.make_async_copy(obuf1_v, out_hbm.at[base + WPB - 1], osem1).wait()


@jax.jit
def _sc_call(allidx, emb):
    mesh = plsc.VectorSubcoreMesh(
        core_axis_name="c", subcore_axis_name="s",
        num_cores=NUM_CORES, num_subcores=NUM_SUBCORES)
    return pl.kernel(
        _sc_body,
        out_type=jax.ShapeDtypeStruct((B, LK, 16), jnp.float32),
        mesh=mesh,
        compiler_params=pltpu.CompilerParams(
            needs_layout_passes=False, use_tc_tiling_on_sc=False),
        scratch_types=[
            pltpu.VMEM((2, IDX_CHUNK, GC, GW), jnp.int32),
            pltpu.VMEM((ROWS, DIM), jnp.float32),
            pltpu.VMEM((ROWS, DIM), jnp.float32),
            pltpu.VMEM((LK, 16), jnp.float32),
            pltpu.VMEM((LK, 16), jnp.float32),
            pltpu.SemaphoreType.DMA,
            pltpu.SemaphoreType.DMA,
            pltpu.SemaphoreType.DMA,
            pltpu.SemaphoreType.DMA,
        ],
    )(allidx, emb)


def kernel(walk, negative, emb):
    allidx = jnp.concatenate(
        [walk, negative.reshape(B, NEG_N)], axis=1).reshape(B, GC, GW)
    out = _sc_call(allidx, emb)
    pos = out[:, :, :K]
    neg = out[:, :, K:K + M]
    return pos, neg
